# baseline (speedup 1.0000x reference)
import functools

import jax
import jax.numpy as jnp
from jax import lax
from jax.experimental import pallas as pl
from jax.experimental.pallas import tpu as pltpu
from jax.experimental.pallas import tpu_sc as plsc

N = 10000
E = 320000
D = 128
H = D // 2

NC = 2
NS = 16

CHUNK = 128
CPT = 160
EPT = CPT * CHUNK
E_PAD = NS * EPT
ACC_N = 10240
ROWS_PER_TILE = ACC_N // NS
DUMMY_DST = N + 100
NBUF = 4

_mesh = plsc.VectorSubcoreMesh(core_axis_name="c", subcore_axis_name="s")


@functools.partial(
    pl.kernel,
    out_type=jax.ShapeDtypeStruct((NC, ACC_N, H), jnp.float32),
    mesh=_mesh,
    scratch_types=[
        pltpu.VMEM((CPT, CHUNK), jnp.int32),
        pltpu.VMEM((CPT, CHUNK), jnp.int32),
        pltpu.VMEM((NBUF, CHUNK, H), jnp.float32),
        pltpu.VMEM_SHARED((ACC_N, H), jnp.float32),
        pltpu.SemaphoreType.DMA((NBUF,)),
        pltpu.SemaphoreType.DMA((NBUF,)),
    ],
    compiler_params=pltpu.CompilerParams(use_tc_tiling_on_sc=False),
)
def _sc_segment_sum(x_hbm, src_hbm, dst_hbm, out_hbm,
                    src_v, dst_v, rows_v, acc_s, gsem, ssem):
    cid = lax.axis_index("c")
    sid = lax.axis_index("s")

    pltpu.sync_copy(src_hbm.at[cid, sid], src_v)
    pltpu.sync_copy(dst_hbm.at[sid], dst_v)

    zv = jnp.zeros((16,), jnp.float32)

    @pl.loop(0, CHUNK)
    def _zero_fill(i):
        for k in range(H // 16):
            rows_v[0, i, pl.ds(k * 16, 16)] = zv

    for t in range(ROWS_PER_TILE // CHUNK):
        pltpu.sync_copy(rows_v.at[0],
                        acc_s.at[pl.ds(sid * ROWS_PER_TILE + t * CHUNK, CHUNK)])
    plsc.subcore_barrier()

    def _gather_start(j, b):
        pltpu.async_copy(x_hbm.at[src_v.at[j]], rows_v.at[b], gsem.at[b])

    def _gather_wait(b):
        pltpu.make_async_copy(x_hbm.at[pl.ds(0, CHUNK)], rows_v.at[b],
                              gsem.at[b]).wait()

    def _scatter_start(j, b):
        pltpu.async_copy(rows_v.at[b], acc_s.at[dst_v.at[j]], ssem.at[b],
                         add=True)

    def _scatter_wait(b):
        pltpu.make_async_copy(rows_v.at[b], acc_s.at[pl.ds(0, CHUNK)],
                              ssem.at[b]).wait()

    _gather_start(0, 0)
    _gather_start(1, 1)

    @pl.loop(0, CPT, step=NBUF)
    def _edges(jv):
        for u in range(NBUF):
            j = jv + u

            @pl.when(j >= 2)
            def _():
                _scatter_wait((u + 2) % NBUF)

            @pl.when(j + 2 < CPT)
            def _():
                _gather_start(j + 2, (u + 2) % NBUF)

            _gather_wait(u)
            _scatter_start(j, u)

    _scatter_wait((CPT - 2) % NBUF)
    _scatter_wait((CPT - 1) % NBUF)

    plsc.subcore_barrier()

    pltpu.sync_copy(acc_s.at[pl.ds(sid * ROWS_PER_TILE, ROWS_PER_TILE)],
                    out_hbm.at[cid, pl.ds(sid * ROWS_PER_TILE, ROWS_PER_TILE)])


_TC_BLOCK = 2000


def _mlp_body(eps_ref, x_ref, s_ref, w1_ref, b1_ref, w2_ref, b2_ref, o_ref):
    s = jnp.concatenate([s_ref[0], s_ref[1]], axis=-1)
    z = (1.0 + eps_ref[0]) * x_ref[...] + s
    h = jnp.maximum(
        jnp.dot(z, w1_ref[...], preferred_element_type=jnp.float32) + b1_ref[...],
        0.0)
    o_ref[...] = (
        jnp.dot(h, w2_ref[...], preferred_element_type=jnp.float32) + b2_ref[...])


def _tc_mlp(x, s, eps, w1, b1, w2, b2):
    return pl.pallas_call(
        _mlp_body,
        grid=(N // _TC_BLOCK,),
        in_specs=[
            pl.BlockSpec(memory_space=pltpu.SMEM),
            pl.BlockSpec((_TC_BLOCK, D), lambda i: (i, 0)),
            pl.BlockSpec((NC, _TC_BLOCK, H), lambda i: (0, i, 0)),
            pl.BlockSpec((D, D), lambda i: (0, 0)),
            pl.BlockSpec((1, D), lambda i: (0, 0)),
            pl.BlockSpec((D, D), lambda i: (0, 0)),
            pl.BlockSpec((1, D), lambda i: (0, 0)),
        ],
        out_specs=pl.BlockSpec((_TC_BLOCK, D), lambda i: (i, 0)),
        out_shape=jax.ShapeDtypeStruct((N, D), jnp.float32),
    )(eps, x, s, w1, b1, w2, b2)


def kernel(X, edge_index, eps_0, W1_0, b1_0, W2_0, b2_0,
           eps_1, W1_1, b1_1, W2_1, b2_1,
           eps_2, W1_2, b1_2, W2_2, b2_2):
    pad = E_PAD - E
    srcp = jnp.pad(edge_index[0], (0, pad))
    dstp = jnp.pad(edge_index[1], (0, pad), constant_values=DUMMY_DST)
    src2 = 2 * srcp
    src = jnp.stack([src2, src2 + 1]).reshape(NC, NS, CPT, CHUNK)
    dst = dstp.reshape(NS, CPT, CHUNK)
    params = [
        (eps_0, W1_0, b1_0, W2_0, b2_0),
        (eps_1, W1_1, b1_1, W2_1, b2_1),
        (eps_2, W1_2, b1_2, W2_2, b2_2),
    ]
    x = X
    for (eps, w1, b1, w2, b2) in params:
        s = _sc_segment_sum(x.reshape(2 * N, H), src, dst)
        x = _tc_mlp(x, s, eps, w1, b1.reshape(1, D), w2, b2.reshape(1, D))
    return x
